# baseline (speedup 1.0000x reference)
import functools

import jax
import jax.numpy as jnp
from jax import lax
from jax.experimental import pallas as pl
from jax.experimental.pallas import tpu as pltpu
from jax.experimental.pallas import tpu_sc as plsc

_VOCAB = 100000
_D = 1024
_B = 4096

_NC = 2
_NS = 16
_NW = _NC * _NS
_L = 16

_BPW = _B // _NW
_G = 16
_NCH = _BPW // _G


def _prep_body(W1a_ref, W1b_ref, W2a_ref, W2b_ref,
               b1a_ref, b1b_ref, b2a_ref, b2b_ref,
               v1_ref, v12_ref, c_ref):
    v1 = jnp.dot(W1a_ref[...], W1b_ref[...], preferred_element_type=jnp.float32)
    v2 = jnp.dot(W2a_ref[...], W2b_ref[...], preferred_element_type=jnp.float32)
    v1_ref[...] = v1
    v12_ref[...] = v1 + v2
    c = (jnp.dot(b1a_ref[...], W1b_ref[...], preferred_element_type=jnp.float32)
         + b1b_ref[...]
         + jnp.dot(b2a_ref[...], W2b_ref[...], preferred_element_type=jnp.float32)
         + b2b_ref[...])
    c_ref[...] = jnp.broadcast_to(c, c_ref.shape)


_prep = pl.pallas_call(
    _prep_body,
    out_shape=[
        jax.ShapeDtypeStruct((_D, 1), jnp.float32),
        jax.ShapeDtypeStruct((_D, 1), jnp.float32),
        jax.ShapeDtypeStruct((1, 128), jnp.float32),
    ],
)


def _sc_body(x_hbm, t1_hbm, t2_hbm, v1_hbm, v12_hbm, c_hbm,
             out_hbm,
             idx_v, v1_v, v12_v, c_v, buf1, buf2, outb,
             s1a, s1b, s2a, s2b):
    wid = lax.axis_index("s") * _NC + lax.axis_index("c")
    base = wid * _BPW
    pltpu.sync_copy(x_hbm.at[pl.ds(base, _BPW)], idx_v)
    pltpu.sync_copy(v1_hbm, v1_v)
    pltpu.sync_copy(v12_hbm, v12_v)
    pltpu.sync_copy(c_hbm, c_v)

    sems1 = (s1a, s1b)
    sems2 = (s2a, s2b)

    def start(k):
        s = k % 2
        isl = idx_v.at[pl.ds(k * _G, _G)]
        h1 = pltpu.async_copy(t1_hbm.at[isl], buf1.at[s], sems1[s])
        h2 = pltpu.async_copy(t2_hbm.at[isl], buf2.at[s], sems2[s])
        return h1, h2

    rows16 = lax.iota(jnp.int32, _L)
    handles = [None, None]
    handles[0] = start(0)
    for k in range(_NCH):
        s = k % 2
        if k + 1 < _NCH:
            handles[(k + 1) % 2] = start(k + 1)
        h1, h2 = handles[s]
        h1.wait()
        h2.wait()
        b1 = buf1.at[s]
        b2 = buf2.at[s]

        def jbody(j, accs):
            a1, a2 = accs
            v1c = v1_v[pl.ds(j * _L, _L)]
            v12c = v12_v[pl.ds(j * _L, _L)]
            jj = jnp.full((_L,), j * _L, jnp.int32)
            for l in range(_L):
                col1 = plsc.load_gather(b1, [rows16, jj + l])
                col2 = plsc.load_gather(b2, [rows16, jj + l])
                a1 = a1 + col1 * v1c[l]
                a2 = a2 + col2 * v12c[l]
            return (a1, a2)

        a1, a2 = lax.fori_loop(0, _D // _L, jbody,
                               (c_v[...], jnp.zeros((_L,), jnp.float32)))
        outb[pl.ds(k * _G, _G)] = a1 + a2
    pltpu.sync_copy(outb, out_hbm.at[pl.ds(base, _BPW)])


_sc = functools.partial(
    pl.kernel,
    mesh=plsc.VectorSubcoreMesh(core_axis_name="c", subcore_axis_name="s"),
    compiler_params=pltpu.CompilerParams(use_tc_tiling_on_sc=False,
                                         needs_layout_passes=False),
    out_type=jax.ShapeDtypeStruct((_B,), jnp.float32),
    scratch_types=[
        pltpu.VMEM((_BPW,), jnp.int32),
        pltpu.VMEM((_D,), jnp.float32),
        pltpu.VMEM((_D,), jnp.float32),
        pltpu.VMEM((_L,), jnp.float32),
        pltpu.VMEM((2, _G, _D), jnp.float32),
        pltpu.VMEM((2, _G, _D), jnp.float32),
        pltpu.VMEM((_BPW,), jnp.float32),
        pltpu.SemaphoreType.DMA,
        pltpu.SemaphoreType.DMA,
        pltpu.SemaphoreType.DMA,
        pltpu.SemaphoreType.DMA,
    ],
)(_sc_body)


def kernel(x, table_1, table_2, W1a, b1a, W1b, b1b, W2a, b2a, W2b, b2b):
    v1c, v12c, csp = _prep(W1a, W1b, W2a, W2b,
                           b1a.reshape(1, -1), b1b.reshape(1, 1),
                           b2a.reshape(1, -1), b2b.reshape(1, 1))
    out = _sc(x, table_1, table_2,
              v1c.reshape(_D), v12c.reshape(_D), csp.reshape(128)[:_L])
    return out.reshape(_B, 1)


# speedup vs baseline: 3.5167x; 3.5167x over previous
import functools

import jax
import jax.numpy as jnp
from jax import lax
from jax.experimental import pallas as pl
from jax.experimental.pallas import tpu as pltpu
from jax.experimental.pallas import tpu_sc as plsc

_VOCAB = 100000
_D = 1024
_B = 4096

_NC = 2
_NS = 16
_NW = _NC * _NS
_L = 16

_BPW = _B // _NW
_G = 16
_NCH = _BPW // _G


def _prep_body(W1a_ref, W1b_ref, W2a_ref, W2b_ref,
               b1a_ref, b1b_ref, b2a_ref, b2b_ref,
               v1_ref, v12_ref, c_ref):
    v1 = jnp.dot(W1a_ref[...], W1b_ref[...], preferred_element_type=jnp.float32)
    v2 = jnp.dot(W2a_ref[...], W2b_ref[...], preferred_element_type=jnp.float32)
    v1_ref[...] = v1
    v12_ref[...] = v1 + v2
    c = (jnp.dot(b1a_ref[...], W1b_ref[...], preferred_element_type=jnp.float32)
         + b1b_ref[...]
         + jnp.dot(b2a_ref[...], W2b_ref[...], preferred_element_type=jnp.float32)
         + b2b_ref[...])
    c_ref[...] = jnp.broadcast_to(c, c_ref.shape)


_prep = pl.pallas_call(
    _prep_body,
    out_shape=[
        jax.ShapeDtypeStruct((_D, 1), jnp.float32),
        jax.ShapeDtypeStruct((_D, 1), jnp.float32),
        jax.ShapeDtypeStruct((1, 128), jnp.float32),
    ],
)


def _sc_body(x_hbm, t1_hbm, t2_hbm, v1_hbm, v12_hbm, c_hbm,
             out_hbm,
             idx_v, v1_v, v12_v, c_v, buf1, buf2, outb,
             s1a, s1b, s2a, s2b):
    wid = lax.axis_index("s") * _NC + lax.axis_index("c")
    base = wid * _BPW
    pltpu.sync_copy(x_hbm.at[pl.ds(base, _BPW)], idx_v)
    pltpu.sync_copy(v1_hbm, v1_v)
    pltpu.sync_copy(v12_hbm, v12_v)
    pltpu.sync_copy(c_hbm, c_v)

    sems1 = (s1a, s1b)
    sems2 = (s2a, s2b)

    def start(k):
        s = k % 2
        isl = idx_v.at[pl.ds(k * _G, _G)]
        h1 = pltpu.async_copy(t1_hbm.at[isl], buf1.at[s], sems1[s])
        h2 = pltpu.async_copy(t2_hbm.at[isl], buf2.at[s], sems2[s])
        return h1, h2

    rows16 = lax.iota(jnp.int32, _L)
    handles = [None, None]
    handles[0] = start(0)
    for k in range(_NCH):
        s = k % 2
        if k + 1 < _NCH:
            handles[(k + 1) % 2] = start(k + 1)
        h1, h2 = handles[s]
        h1.wait()
        h2.wait()
        b1 = buf1.at[s]
        b2 = buf2.at[s]

        def jbody(j, accs):
            a1, a2 = accs
            v1c = v1_v[pl.ds(j * _L, _L)]
            v12c = v12_v[pl.ds(j * _L, _L)]
            jj = jnp.full((_L,), j * _L, jnp.int32)
            for l in range(_L):
                col1 = plsc.load_gather(b1, [rows16, jj + l])
                col2 = plsc.load_gather(b2, [rows16, jj + l])
                a1 = a1 + col1 * v1c[l]
                a2 = a2 + col2 * v12c[l]
            return (a1, a2)

        a1, a2 = lax.fori_loop(0, _D // _L, jbody,
                               (c_v[...], jnp.zeros((_L,), jnp.float32)))
        outb[pl.ds(k * _G, _G)] = a1 + a2
    pltpu.sync_copy(outb, out_hbm.at[pl.ds(base, _BPW)])


_sc = functools.partial(
    pl.kernel,
    mesh=plsc.VectorSubcoreMesh(core_axis_name="c", subcore_axis_name="s"),
    compiler_params=pltpu.CompilerParams(use_tc_tiling_on_sc=True,
                                         needs_layout_passes=False),
    out_type=jax.ShapeDtypeStruct((_B,), jnp.float32),
    scratch_types=[
        pltpu.VMEM((_BPW,), jnp.int32),
        pltpu.VMEM((_D,), jnp.float32),
        pltpu.VMEM((_D,), jnp.float32),
        pltpu.VMEM((_L,), jnp.float32),
        pltpu.VMEM((2, _G, _D), jnp.float32),
        pltpu.VMEM((2, _G, _D), jnp.float32),
        pltpu.VMEM((_BPW,), jnp.float32),
        pltpu.SemaphoreType.DMA,
        pltpu.SemaphoreType.DMA,
        pltpu.SemaphoreType.DMA,
        pltpu.SemaphoreType.DMA,
    ],
)(_sc_body)


def kernel(x, table_1, table_2, W1a, b1a, W1b, b1b, W2a, b2a, W2b, b2b):
    v1c, v12c, csp = _prep(W1a, W1b, W2a, W2b,
                           b1a.reshape(1, -1), b1b.reshape(1, 1),
                           b2a.reshape(1, -1), b2b.reshape(1, 1))
    out = _sc(x, table_1, table_2,
              v1c.reshape(_D), v12c.reshape(_D), csp.reshape(128)[:_L])
    return out.reshape(_B, 1)
